# baseline (speedup 1.0000x reference)
import functools

import jax
import jax.numpy as jnp
from jax import lax
from jax.experimental import pallas as pl
from jax.experimental.pallas import tpu as pltpu
from jax.experimental.pallas import tpu_sc as plsc

N_NODES = 100000
D = 128
NSEG = 256
NW = 32
CHUNK = 160
CPW = 20
RPW = CHUNK * CPW
NTAIL = N_NODES % RPW
CTAIL = N_NODES % CHUNK
L = 16
NJ = D // L
SUB = 16
GPC = CHUNK // SUB
GPW = RPW // SUB


def _sc_partial_sums(x_flat, batch):
    mesh = plsc.VectorSubcoreMesh(core_axis_name="c", subcore_axis_name="s")

    @functools.partial(
        pl.kernel,
        mesh=mesh,
        out_type=jax.ShapeDtypeStruct((NW, (NSEG + 1) * D), jnp.float32),
        scratch_types=[
            pltpu.VMEM((RPW + SUB,), jnp.int32),
            pltpu.VMEM((CHUNK * D,), jnp.float32),
            pltpu.VMEM((CHUNK * D,), jnp.float32),
            pltpu.VMEM(((NSEG + 1) * D,), jnp.float32),
            pltpu.SemaphoreType.DMA,
            pltpu.SemaphoreType.DMA,
            pltpu.SemaphoreType.DMA,
        ],
    )
    def k(x_hbm, b_hbm, out_hbm, idx_v, rows_a, rows_b, acc_v,
          sem_a, sem_b, sem_i):
        wid = lax.axis_index("s") * 2 + lax.axis_index("c")
        zero16 = jnp.zeros((L,), jnp.float32)
        slab = wid * RPW

        full_slab = slab + RPW <= N_NODES

        def ids_full_dma():
            return pltpu.make_async_copy(
                b_hbm.at[pl.ds(slab, RPW)], idx_v.at[pl.ds(0, RPW)], sem_i)

        def ids_tail_dma():
            return pltpu.make_async_copy(
                b_hbm.at[pl.ds(slab, NTAIL)], idx_v.at[pl.ds(0, NTAIL)],
                sem_i)

        def chunk_dma(c, buf, sem):
            row_base = slab + c * CHUNK
            return pltpu.make_async_copy(
                x_hbm.at[pl.ds(row_base * D, CHUNK * D)], buf, sem)

        def is_real(c):
            return slab + c * CHUNK + CHUNK <= N_NODES

        def tail_dma(c, buf, sem):
            row_base = slab + c * CHUNK
            return pltpu.make_async_copy(
                x_hbm.at[pl.ds(row_base * D, CTAIL * D)],
                buf.at[pl.ds(0, CTAIL * D)], sem)

        def uniform_group(rows_v, row0, first, nrows=SUB):
            base = first * D
            totals = []
            for j in range(NJ):
                stack = []
                for r in range(nrows):
                    v = rows_v[pl.ds(row0 + r * D + L * j, L)]
                    node = (0, v)
                    while stack and stack[-1][0] == node[0]:
                        prank, pv = stack.pop()
                        node = (prank + 1, pv + node[1])
                    stack.append(node)
                total = stack[0][1]
                for _, sv in stack[1:]:
                    total = total + sv
                totals.append(total)
            for j in range(NJ):
                plsc.addupdate(acc_v.at[pl.ds(base + L * j, L)], totals[j])

        def mixed_group(rows_v, row0, gbase):
            segs = idx_v[pl.ds(gbase, L)]
            prev = None
            for r in range(SUB):
                base_r = segs[r] * D
                vals = [rows_v[pl.ds(row0 + r * D + L * j, L)]
                        for j in range(NJ)]
                if prev is not None:
                    pvals, pbase = prev
                    for j in range(NJ):
                        plsc.addupdate(
                            acc_v.at[pl.ds(pbase + L * j, L)], pvals[j])
                prev = (vals, base_r)
            pvals, pbase = prev
            for j in range(NJ):
                plsc.addupdate(acc_v.at[pl.ds(pbase + L * j, L)], pvals[j])

        def pair_body(cp, carry):
            for b, (buf, sem) in enumerate(((rows_a, sem_a),
                                            (rows_b, sem_b))):
                c = 2 * cp + b

                row_base = slab + c * CHUNK
                is_tail = jnp.logical_and(row_base < N_NODES,
                                          row_base + CHUNK > N_NODES)

                @pl.when(is_real(c))
                def _wait():
                    chunk_dma(c, buf, sem).wait()

                if CTAIL:
                    @pl.when(is_tail)
                    def _wait_tail():
                        tail_dma(c, buf, sem).wait()

                cf = idx_v[pl.ds(c * CHUNK, L)][0]
                cl = idx_v[pl.ds(c * CHUNK + CHUNK - L, L)][L - 1]

                @pl.when(cf == cl)
                def _mega():
                    uniform_group(buf, 0, cf, nrows=CHUNK)

                def gb(g, st):
                    grp = c * GPC + g
                    segs = idx_v[pl.ds(grp * SUB, L)]
                    first = segs[0]
                    last = segs[L - 1]
                    row0 = g * SUB * D

                    @pl.when(first == last)
                    def _u():
                        uniform_group(buf, row0, first)

                    @pl.when(first != last)
                    def _m():
                        mixed_group(buf, row0, grp * SUB)

                    return st

                @pl.when(cf != cl)
                def _groups():
                    lax.fori_loop(0, GPC, gb, 0)

                nrb = slab + (c + 2) * CHUNK

                @pl.when(jnp.logical_and(is_real(c + 2), c + 2 < CPW))
                def _start():
                    chunk_dma(c + 2, buf, sem).start()

                if CTAIL:
                    @pl.when(jnp.logical_and(
                        jnp.logical_and(nrb < N_NODES,
                                        nrb + CHUNK > N_NODES),
                        c + 2 < CPW))
                    def _start_tail():
                        tail_dma(c + 2, buf, sem).start()

            return carry

        @pl.when(is_real(0))
        def _p0():
            chunk_dma(0, rows_a, sem_a).start()

        @pl.when(is_real(1))
        def _p1():
            chunk_dma(1, rows_b, sem_b).start()

        def prime_tail(c, buf, sem):
            row_base = slab + c * CHUNK

            @pl.when(jnp.logical_and(row_base < N_NODES,
                                     row_base + CHUNK > N_NODES))
            def _pt():
                tail_dma(c, buf, sem).start()

        if CTAIL:
            prime_tail(0, rows_a, sem_a)
            prime_tail(1, rows_b, sem_b)

        @pl.when(full_slab)
        def _ids_full():
            ids_full_dma().start()

        @pl.when(jnp.logical_not(full_slab))
        def _ids_tail():
            ids_tail_dma().start()
            pad16 = jnp.full((L,), NSEG, jnp.int32)

            def prow(i, carry):
                idx_v[pl.ds(i * L, L)] = pad16
                return carry

            lax.fori_loop(NTAIL // L, (RPW + SUB) // L, prow, 0)

        def zrow(i, carry):
            for j in range(NJ):
                acc_v[pl.ds(i * D + L * j, L)] = zero16
            return carry

        lax.fori_loop(0, NSEG + 1, zrow, 0)

        @pl.when(full_slab)
        def _ids_full_w():
            ids_full_dma().wait()

        @pl.when(jnp.logical_not(full_slab))
        def _ids_tail_w():
            ids_tail_dma().wait()

        lax.fori_loop(0, CPW // 2, pair_body, 0)

        pltpu.sync_copy(acc_v, out_hbm.at[wid])

    return k(x_flat, batch)


def _tc_reduce(partials):
    def body(p_ref, o_ref):
        p = p_ref[...].reshape(NW, NSEG + 1, D)
        o_ref[...] = jnp.sum(p[:, :NSEG, :], axis=0)

    return pl.pallas_call(
        body,
        out_shape=jax.ShapeDtypeStruct((NSEG, D), jnp.float32),
    )(partials)


def kernel(x, batch):
    partials = _sc_partial_sums(x.reshape(-1), batch)
    return _tc_reduce(partials)


# speedup vs baseline: 2.4807x; 2.4807x over previous
import functools

import jax
import jax.numpy as jnp
from jax import lax
from jax.experimental import pallas as pl
from jax.experimental.pallas import tpu as pltpu
from jax.experimental.pallas import tpu_sc as plsc

N_NODES = 100000
D = 128
NSEG = 256
NW = 32
CHUNK = 160
CPW = 20
RPW = CHUNK * CPW
NTAIL = N_NODES % RPW
CTAIL = N_NODES % CHUNK
L = 16
NJ = D // L
SUB = 16
GPC = CHUNK // SUB
GPW = RPW // SUB


def _sc_partial_sums(x_flat, batch):
    mesh = plsc.VectorSubcoreMesh(core_axis_name="c", subcore_axis_name="s")

    @functools.partial(
        pl.kernel,
        mesh=mesh,
        out_type=jax.ShapeDtypeStruct((NW, (NSEG + 1) * D), jnp.float32),
        scratch_types=[
            pltpu.VMEM((RPW + SUB,), jnp.int32),
            pltpu.VMEM((CHUNK * D,), jnp.float32),
            pltpu.VMEM((CHUNK * D,), jnp.float32),
            pltpu.VMEM(((NSEG + 1) * D,), jnp.float32),
            pltpu.SemaphoreType.DMA,
            pltpu.SemaphoreType.DMA,
            pltpu.SemaphoreType.DMA,
        ],
    )
    def k(x_hbm, b_hbm, out_hbm, idx_v, rows_a, rows_b, acc_v,
          sem_a, sem_b, sem_i):
        wid = lax.axis_index("s") * 2 + lax.axis_index("c")
        zero16 = jnp.zeros((L,), jnp.float32)
        slab = wid * RPW

        full_slab = slab + RPW <= N_NODES

        def ids_full_dma():
            return pltpu.make_async_copy(
                b_hbm.at[pl.ds(slab, RPW)], idx_v.at[pl.ds(0, RPW)], sem_i)

        def ids_tail_dma():
            return pltpu.make_async_copy(
                b_hbm.at[pl.ds(slab, NTAIL)], idx_v.at[pl.ds(0, NTAIL)],
                sem_i)

        def chunk_dma(c, buf, sem):
            row_base = slab + c * CHUNK
            return pltpu.make_async_copy(
                x_hbm.at[pl.ds(row_base * D, CHUNK * D)], buf, sem)

        def is_real(c):
            return slab + c * CHUNK + CHUNK <= N_NODES

        def tail_dma(c, buf, sem):
            row_base = slab + c * CHUNK
            return pltpu.make_async_copy(
                x_hbm.at[pl.ds(row_base * D, CTAIL * D)],
                buf.at[pl.ds(0, CTAIL * D)], sem)

        def uniform_group(rows_v, row0, first):
            base = first * D
            totals = []
            for j in range(NJ):
                stack = []
                for r in range(SUB):
                    v = rows_v[pl.ds(row0 + r * D + L * j, L)]
                    node = (0, v)
                    while stack and stack[-1][0] == node[0]:
                        prank, pv = stack.pop()
                        node = (prank + 1, pv + node[1])
                    stack.append(node)
                total = stack[0][1]
                for _, sv in stack[1:]:
                    total = total + sv
                totals.append(total)
            for j in range(NJ):
                plsc.addupdate(acc_v.at[pl.ds(base + L * j, L)], totals[j])

        def mixed_group(rows_v, row0, gbase):
            segs = idx_v[pl.ds(gbase, L)]
            prev = None
            for r in range(SUB):
                base_r = segs[r] * D
                vals = [rows_v[pl.ds(row0 + r * D + L * j, L)]
                        for j in range(NJ)]
                if prev is not None:
                    pvals, pbase = prev
                    for j in range(NJ):
                        plsc.addupdate(
                            acc_v.at[pl.ds(pbase + L * j, L)], pvals[j])
                prev = (vals, base_r)
            pvals, pbase = prev
            for j in range(NJ):
                plsc.addupdate(acc_v.at[pl.ds(pbase + L * j, L)], pvals[j])

        def pair_body(cp, carry):
            for b, (buf, sem) in enumerate(((rows_a, sem_a),
                                            (rows_b, sem_b))):
                c = 2 * cp + b

                row_base = slab + c * CHUNK
                is_tail = jnp.logical_and(row_base < N_NODES,
                                          row_base + CHUNK > N_NODES)

                @pl.when(is_real(c))
                def _wait():
                    chunk_dma(c, buf, sem).wait()

                if CTAIL:
                    @pl.when(is_tail)
                    def _wait_tail():
                        tail_dma(c, buf, sem).wait()

                def gb(g, st):
                    grp = c * GPC + g
                    segs = idx_v[pl.ds(grp * SUB, L)]
                    first = segs[0]
                    last = segs[L - 1]
                    row0 = g * SUB * D

                    @pl.when(first == last)
                    def _u():
                        uniform_group(buf, row0, first)

                    @pl.when(first != last)
                    def _m():
                        mixed_group(buf, row0, grp * SUB)

                    return st

                carry = lax.fori_loop(0, GPC, gb, carry)

                nrb = slab + (c + 2) * CHUNK

                @pl.when(jnp.logical_and(is_real(c + 2), c + 2 < CPW))
                def _start():
                    chunk_dma(c + 2, buf, sem).start()

                if CTAIL:
                    @pl.when(jnp.logical_and(
                        jnp.logical_and(nrb < N_NODES,
                                        nrb + CHUNK > N_NODES),
                        c + 2 < CPW))
                    def _start_tail():
                        tail_dma(c + 2, buf, sem).start()

            return carry

        @pl.when(is_real(0))
        def _p0():
            chunk_dma(0, rows_a, sem_a).start()

        @pl.when(is_real(1))
        def _p1():
            chunk_dma(1, rows_b, sem_b).start()

        def prime_tail(c, buf, sem):
            row_base = slab + c * CHUNK

            @pl.when(jnp.logical_and(row_base < N_NODES,
                                     row_base + CHUNK > N_NODES))
            def _pt():
                tail_dma(c, buf, sem).start()

        if CTAIL:
            prime_tail(0, rows_a, sem_a)
            prime_tail(1, rows_b, sem_b)

        @pl.when(full_slab)
        def _ids_full():
            ids_full_dma().start()

        @pl.when(jnp.logical_not(full_slab))
        def _ids_tail():
            ids_tail_dma().start()
            pad16 = jnp.full((L,), NSEG, jnp.int32)

            def prow(i, carry):
                idx_v[pl.ds(i * L, L)] = pad16
                return carry

            lax.fori_loop(NTAIL // L, (RPW + SUB) // L, prow, 0)

        def zrow(i, carry):
            for j in range(NJ):
                acc_v[pl.ds(i * D + L * j, L)] = zero16
            return carry

        lax.fori_loop(0, NSEG + 1, zrow, 0)

        @pl.when(full_slab)
        def _ids_full_w():
            ids_full_dma().wait()

        @pl.when(jnp.logical_not(full_slab))
        def _ids_tail_w():
            ids_tail_dma().wait()

        lax.fori_loop(0, CPW // 2, pair_body, 0)

        pltpu.sync_copy(acc_v, out_hbm.at[wid])

    return k(x_flat, batch)


def _tc_reduce(partials):
    def body(p_ref, o_ref):
        p = p_ref[...].reshape(NW, NSEG + 1, D)
        o_ref[...] = jnp.sum(p[:, :NSEG, :], axis=0)

    return pl.pallas_call(
        body,
        out_shape=jax.ShapeDtypeStruct((NSEG, D), jnp.float32),
    )(partials)


def kernel(x, batch):
    partials = _sc_partial_sums(x.reshape(-1), batch)
    return _tc_reduce(partials)
